# baseline (device time: 28405 ns/iter reference)
import jax
import jax.numpy as jnp
from jax import lax
from jax.experimental import pallas as pl
from jax.experimental.pallas import tpu as pltpu

N_DEV = 32
NZ = 4
NP = 8


def kernel(x, w_mat):
    k_dim, k_per = x.shape
    n = w_mat.shape[1]
    m_per = k_dim // N_DEV
    half = m_per // 2

    def body(x_hbm, w_hbm, out_hbm, xv_ref, xpack_ref, g1_ref, g2_ref,
             g3_ref, wbuf_ref, out_ref, p1_send, p1_recv, p2_send, p2_recv,
             w_sems, x_sem, out_sem):
        my_i = lax.axis_index("i")
        my_z = my_i // NP
        my_p = lax.rem(my_i, NP)

        barrier_sem = pltpu.get_barrier_semaphore()
        pl.semaphore_signal(barrier_sem, 1)
        pl.semaphore_wait(barrier_sem, 1)

        for j in range(N_DEV):
            pltpu.make_async_copy(
                w_hbm.at[pl.ds(j * m_per, m_per), :],
                wbuf_ref.at[j],
                w_sems.at[j],
            ).start()

        xcopy = pltpu.make_async_copy(x_hbm, xv_ref, x_sem)
        xcopy.start()
        xcopy.wait()
        for pt in range(NP):
            for zt in range(NZ):
                j = NP * zt + pt
                xpack_ref[pt, zt] = jnp.concatenate(
                    [xv_ref[pl.ds(j * m_per, half), :],
                     xv_ref[pl.ds(j * m_per + half, half), :]],
                    axis=1,
                )

        g1_ref[my_p] = xpack_ref[my_p]

        sends1 = []
        for dp in range(1, NP):
            pt = lax.rem(my_p + dp, NP)
            peer = NP * my_z + pt
            rdma = pltpu.make_async_remote_copy(
                src_ref=xpack_ref.at[pt],
                dst_ref=g1_ref.at[my_p],
                send_sem=p1_send.at[dp],
                recv_sem=p1_recv.at[my_p],
                device_id=(peer,),
                device_id_type=pl.DeviceIdType.MESH,
            )
            rdma.start()
            sends1.append(rdma)

        for sp in range(NP):
            @pl.when(sp != my_p)
            def _():
                recv = pltpu.make_async_remote_copy(
                    src_ref=g1_ref.at[sp],
                    dst_ref=g1_ref.at[sp],
                    send_sem=p1_send.at[0],
                    recv_sem=p1_recv.at[sp],
                    device_id=(my_p,),
                    device_id_type=pl.DeviceIdType.MESH,
                )
                recv.wait_recv()

        for zt in range(NZ):
            for sp in range(NP):
                g2_ref[zt, sp] = g1_ref[sp, zt]

        sends2 = []
        for dz in range(1, NZ):
            zt = lax.rem(my_z + dz, NZ)
            peer = NP * zt + my_p
            rdma = pltpu.make_async_remote_copy(
                src_ref=g2_ref.at[zt],
                dst_ref=g3_ref.at[my_z],
                send_sem=p2_send.at[dz],
                recv_sem=p2_recv.at[my_z],
                device_id=(peer,),
                device_id_type=pl.DeviceIdType.MESH,
            )
            rdma.start()
            sends2.append(rdma)

        for sp in range(NP):
            tile_p = g1_ref[sp, my_z]
            tile = jnp.concatenate(
                [tile_p[:, :k_per], tile_p[:, k_per:]], axis=0)
            j = NP * my_z + sp
            pltpu.make_async_copy(
                w_hbm.at[pl.ds(j * m_per, m_per), :],
                wbuf_ref.at[j],
                w_sems.at[j],
            ).wait()
            part = jnp.dot(tile, wbuf_ref[j],
                           preferred_element_type=jnp.float32)
            if sp == 0:
                out_ref[:, :] = part
            else:
                out_ref[:, :] += part

        for dz in range(1, NZ):
            zp = lax.rem(my_z + dz, NZ)
            recv = pltpu.make_async_remote_copy(
                src_ref=g3_ref.at[zp],
                dst_ref=g3_ref.at[zp],
                send_sem=p2_send.at[0],
                recv_sem=p2_recv.at[zp],
                device_id=(my_z,),
                device_id_type=pl.DeviceIdType.MESH,
            )
            recv.wait_recv()
            for sp in range(NP):
                tile_p = g3_ref[zp, sp]
                tile = jnp.concatenate(
                    [tile_p[:, :k_per], tile_p[:, k_per:]], axis=0)
                j = NP * zp + sp
                pltpu.make_async_copy(
                    w_hbm.at[pl.ds(j * m_per, m_per), :],
                    wbuf_ref.at[j],
                    w_sems.at[j],
                ).wait()
                out_ref[:, :] += jnp.dot(
                    tile, wbuf_ref[j], preferred_element_type=jnp.float32)

        out_ref[:, :] = jnp.maximum(out_ref[:, :], 0.0)

        ocopy = pltpu.make_async_copy(out_ref, out_hbm, out_sem)
        ocopy.start()
        ocopy.wait()

        for rdma in sends1 + sends2:
            rdma.wait_send()

    return pl.pallas_call(
        body,
        out_shape=jax.ShapeDtypeStruct((m_per, n), jnp.float32),
        in_specs=[
            pl.BlockSpec(memory_space=pltpu.MemorySpace.HBM),
            pl.BlockSpec(memory_space=pltpu.MemorySpace.HBM),
        ],
        out_specs=pl.BlockSpec(memory_space=pltpu.MemorySpace.HBM),
        scratch_shapes=[
            pltpu.VMEM((k_dim, k_per), jnp.float32),
            pltpu.VMEM((NP, NZ, half, 2 * k_per), jnp.float32),
            pltpu.VMEM((NP, NZ, half, 2 * k_per), jnp.float32),
            pltpu.VMEM((NZ, NP, half, 2 * k_per), jnp.float32),
            pltpu.VMEM((NZ, NP, half, 2 * k_per), jnp.float32),
            pltpu.VMEM((N_DEV, m_per, n), jnp.float32),
            pltpu.VMEM((m_per, n), jnp.float32),
            pltpu.SemaphoreType.DMA((NP,)),
            pltpu.SemaphoreType.DMA((NP,)),
            pltpu.SemaphoreType.DMA((NZ,)),
            pltpu.SemaphoreType.DMA((NZ,)),
            pltpu.SemaphoreType.DMA((N_DEV,)),
            pltpu.SemaphoreType.DMA,
            pltpu.SemaphoreType.DMA,
        ],
        compiler_params=pltpu.CompilerParams(collective_id=0),
    )(
        pltpu.with_memory_space_constraint(x, pltpu.MemorySpace.HBM),
        pltpu.with_memory_space_constraint(w_mat, pltpu.MemorySpace.HBM),
    )


# device time: 25678 ns/iter; 1.1062x vs baseline; 1.1062x over previous
import jax
import jax.numpy as jnp
from jax import lax
from jax.experimental import pallas as pl
from jax.experimental.pallas import tpu as pltpu

N_DEV = 32


def kernel(x, w_mat):
    k_dim, k_per = x.shape
    n = w_mat.shape[1]
    m_per = k_dim // N_DEV
    half = m_per // 2

    def body(x_hbm, w_hbm, out_hbm, xv_ref, xpack_ref, gpack_ref, wbuf_ref,
             out_ref, send_sems, recv_sems, w_sems, x_sem, out_sem):
        my_i = lax.axis_index("i")

        barrier_sem = pltpu.get_barrier_semaphore()
        pl.semaphore_signal(barrier_sem, 1)
        pl.semaphore_wait(barrier_sem, 1)

        for j in range(N_DEV):
            pltpu.make_async_copy(
                w_hbm.at[pl.ds(j * m_per, m_per), :],
                wbuf_ref.at[j],
                w_sems.at[j],
            ).start()

        xcopy = pltpu.make_async_copy(x_hbm, xv_ref, x_sem)
        xcopy.start()
        xcopy.wait()
        for j in range(N_DEV):
            xpack_ref[j] = jnp.concatenate(
                [xv_ref[pl.ds(j * m_per, half), :],
                 xv_ref[pl.ds(j * m_per + half, half), :]],
                axis=1,
            )

        gpack_ref[my_i] = xpack_ref[my_i]

        sends = []
        for d in range(1, N_DEV):
            j = lax.rem(my_i + d, N_DEV)
            rdma = pltpu.make_async_remote_copy(
                src_ref=xpack_ref.at[j],
                dst_ref=gpack_ref.at[my_i],
                send_sem=send_sems.at[d],
                recv_sem=recv_sems.at[my_i],
                device_id=(j,),
                device_id_type=pl.DeviceIdType.MESH,
            )
            rdma.start()
            sends.append(rdma)

        for d in range(N_DEV):
            j = lax.rem(my_i + d, N_DEV)

            pltpu.make_async_copy(
                w_hbm.at[pl.ds(j * m_per, m_per), :],
                wbuf_ref.at[j],
                w_sems.at[j],
            ).wait()

            if d > 0:
                recv = pltpu.make_async_remote_copy(
                    src_ref=gpack_ref.at[j],
                    dst_ref=gpack_ref.at[j],
                    send_sem=send_sems.at[0],
                    recv_sem=recv_sems.at[j],
                    device_id=(my_i,),
                    device_id_type=pl.DeviceIdType.MESH,
                )
                recv.wait_recv()

            packed = gpack_ref[j]
            tile = jnp.concatenate(
                [packed[:, :k_per], packed[:, k_per:]], axis=0)
            part = jnp.dot(
                tile, wbuf_ref[j],
                preferred_element_type=jnp.float32,
            )
            if d == 0:
                out_ref[:, :] = part
            else:
                out_ref[:, :] += part

        out_ref[:, :] = jnp.maximum(out_ref[:, :], 0.0)

        ocopy = pltpu.make_async_copy(out_ref, out_hbm, out_sem)
        ocopy.start()
        ocopy.wait()

        for rdma in sends:
            rdma.wait_send()

    return pl.pallas_call(
        body,
        out_shape=jax.ShapeDtypeStruct((m_per, n), jnp.float32),
        in_specs=[
            pl.BlockSpec(memory_space=pltpu.MemorySpace.HBM),
            pl.BlockSpec(memory_space=pltpu.MemorySpace.HBM),
        ],
        out_specs=pl.BlockSpec(memory_space=pltpu.MemorySpace.HBM),
        scratch_shapes=[
            pltpu.VMEM((k_dim, k_per), jnp.float32),
            pltpu.VMEM((N_DEV, half, 2 * k_per), jnp.float32),
            pltpu.VMEM((N_DEV, half, 2 * k_per), jnp.float32),
            pltpu.VMEM((N_DEV, m_per, n), jnp.float32),
            pltpu.VMEM((m_per, n), jnp.float32),
            pltpu.SemaphoreType.DMA((N_DEV,)),
            pltpu.SemaphoreType.DMA((N_DEV,)),
            pltpu.SemaphoreType.DMA((N_DEV,)),
            pltpu.SemaphoreType.DMA,
            pltpu.SemaphoreType.DMA,
        ],
        compiler_params=pltpu.CompilerParams(collective_id=0),
    )(
        pltpu.with_memory_space_constraint(x, pltpu.MemorySpace.HBM),
        pltpu.with_memory_space_constraint(w_mat, pltpu.MemorySpace.HBM),
    )


# device time: 25119 ns/iter; 1.1308x vs baseline; 1.0223x over previous
import jax
import jax.numpy as jnp
from jax import lax
from jax.experimental import pallas as pl
from jax.experimental.pallas import tpu as pltpu

N_DEV = 32


def kernel(x, w_mat):
    k_dim, k_per = x.shape
    n = w_mat.shape[1]
    m_per = k_dim // N_DEV
    half = m_per // 2

    def body(x_hbm, w_hbm, out_hbm, xv_ref, xpack_ref, gpack_ref, wbuf_ref,
             out_ref, send_sems, recv_sems, w_sems, x_sem, out_sem):
        my_i = lax.axis_index("i")

        barrier_sem = pltpu.get_barrier_semaphore()
        pl.semaphore_signal(barrier_sem, 1)
        pl.semaphore_wait(barrier_sem, 1)

        for j in range(N_DEV):
            pltpu.make_async_copy(
                w_hbm.at[pl.ds(j * m_per, m_per), :],
                wbuf_ref.at[j],
                w_sems.at[j],
            ).start()

        xcopy = pltpu.make_async_copy(x_hbm, xv_ref, x_sem)
        xcopy.start()
        xcopy.wait()

        sends = []
        for d in range(N_DEV - 1, 0, -1):
            j = lax.rem(my_i + d, N_DEV)
            xpack_ref[j] = jnp.concatenate(
                [xv_ref[pl.ds(j * m_per, half), :],
                 xv_ref[pl.ds(j * m_per + half, half), :]],
                axis=1,
            )
            rdma = pltpu.make_async_remote_copy(
                src_ref=xpack_ref.at[j],
                dst_ref=gpack_ref.at[my_i],
                send_sem=send_sems.at[d],
                recv_sem=recv_sems.at[my_i],
                device_id=(j,),
                device_id_type=pl.DeviceIdType.MESH,
            )
            rdma.start()
            sends.append(rdma)

        gpack_ref[my_i] = jnp.concatenate(
            [xv_ref[pl.ds(my_i * m_per, half), :],
             xv_ref[pl.ds(my_i * m_per + half, half), :]],
            axis=1,
        )

        for d in range(N_DEV):
            j = lax.rem(my_i + d, N_DEV)

            pltpu.make_async_copy(
                w_hbm.at[pl.ds(j * m_per, m_per), :],
                wbuf_ref.at[j],
                w_sems.at[j],
            ).wait()

            if d > 0:
                recv = pltpu.make_async_remote_copy(
                    src_ref=gpack_ref.at[j],
                    dst_ref=gpack_ref.at[j],
                    send_sem=send_sems.at[0],
                    recv_sem=recv_sems.at[j],
                    device_id=(my_i,),
                    device_id_type=pl.DeviceIdType.MESH,
                )
                recv.wait_recv()

            packed = gpack_ref[j]
            tile = jnp.concatenate(
                [packed[:, :k_per], packed[:, k_per:]], axis=0)
            part = jnp.dot(
                tile, wbuf_ref[j],
                preferred_element_type=jnp.float32,
            )
            if d == 0:
                out_ref[:, :] = part
            else:
                out_ref[:, :] += part

        out_ref[:, :] = jnp.maximum(out_ref[:, :], 0.0)

        ocopy = pltpu.make_async_copy(out_ref, out_hbm, out_sem)
        ocopy.start()
        ocopy.wait()

        for rdma in sends:
            rdma.wait_send()

    return pl.pallas_call(
        body,
        out_shape=jax.ShapeDtypeStruct((m_per, n), jnp.float32),
        in_specs=[
            pl.BlockSpec(memory_space=pltpu.MemorySpace.HBM),
            pl.BlockSpec(memory_space=pltpu.MemorySpace.HBM),
        ],
        out_specs=pl.BlockSpec(memory_space=pltpu.MemorySpace.HBM),
        scratch_shapes=[
            pltpu.VMEM((k_dim, k_per), jnp.float32),
            pltpu.VMEM((N_DEV, half, 2 * k_per), jnp.float32),
            pltpu.VMEM((N_DEV, half, 2 * k_per), jnp.float32),
            pltpu.VMEM((N_DEV, m_per, n), jnp.float32),
            pltpu.VMEM((m_per, n), jnp.float32),
            pltpu.SemaphoreType.DMA((N_DEV,)),
            pltpu.SemaphoreType.DMA((N_DEV,)),
            pltpu.SemaphoreType.DMA((N_DEV,)),
            pltpu.SemaphoreType.DMA,
            pltpu.SemaphoreType.DMA,
        ],
        compiler_params=pltpu.CompilerParams(collective_id=0),
    )(
        pltpu.with_memory_space_constraint(x, pltpu.MemorySpace.HBM),
        pltpu.with_memory_space_constraint(w_mat, pltpu.MemorySpace.HBM),
    )


# device time: 25090 ns/iter; 1.1321x vs baseline; 1.0012x over previous
import jax
import jax.numpy as jnp
from jax import lax
from jax.experimental import pallas as pl
from jax.experimental.pallas import tpu as pltpu

N_DEV = 32


def kernel(x, w_mat):
    k_dim, k_per = x.shape
    n = w_mat.shape[1]
    m_per = k_dim // N_DEV
    half = m_per // 2

    def body(x_hbm, w_hbm, out_hbm, xv_ref, xpack_ref, gpack_ref, wbuf_ref,
             out_ref, send_sems, recv_sems, w_sems, x_sem, out_sem):
        my_i = lax.axis_index("i")

        barrier_sem = pltpu.get_barrier_semaphore()
        pl.semaphore_signal(barrier_sem, 1)
        pl.semaphore_wait(barrier_sem, 1)

        for d in range(N_DEV):
            j = lax.rem(my_i + d, N_DEV)
            pltpu.make_async_copy(
                w_hbm.at[pl.ds(j * m_per, m_per), :],
                wbuf_ref.at[j],
                w_sems.at[j],
            ).start()

        xcopy = pltpu.make_async_copy(x_hbm, xv_ref, x_sem)
        xcopy.start()
        xcopy.wait()

        sends = []
        for d in range(N_DEV - 1, 0, -1):
            j = lax.rem(my_i + d, N_DEV)
            xpack_ref[j] = jnp.concatenate(
                [xv_ref[pl.ds(j * m_per, half), :],
                 xv_ref[pl.ds(j * m_per + half, half), :]],
                axis=1,
            )
            rdma = pltpu.make_async_remote_copy(
                src_ref=xpack_ref.at[j],
                dst_ref=gpack_ref.at[my_i],
                send_sem=send_sems.at[d],
                recv_sem=recv_sems.at[my_i],
                device_id=(j,),
                device_id_type=pl.DeviceIdType.MESH,
            )
            rdma.start()
            sends.append(rdma)

        gpack_ref[my_i] = jnp.concatenate(
            [xv_ref[pl.ds(my_i * m_per, half), :],
             xv_ref[pl.ds(my_i * m_per + half, half), :]],
            axis=1,
        )

        for d in range(N_DEV):
            j = lax.rem(my_i + d, N_DEV)

            pltpu.make_async_copy(
                w_hbm.at[pl.ds(j * m_per, m_per), :],
                wbuf_ref.at[j],
                w_sems.at[j],
            ).wait()

            if d > 0:
                recv = pltpu.make_async_remote_copy(
                    src_ref=gpack_ref.at[j],
                    dst_ref=gpack_ref.at[j],
                    send_sem=send_sems.at[0],
                    recv_sem=recv_sems.at[j],
                    device_id=(my_i,),
                    device_id_type=pl.DeviceIdType.MESH,
                )
                recv.wait_recv()

            packed = gpack_ref[j]
            tile = jnp.concatenate(
                [packed[:, :k_per], packed[:, k_per:]], axis=0)
            part = jnp.dot(
                tile, wbuf_ref[j],
                preferred_element_type=jnp.float32,
            )
            if d == 0:
                out_ref[:, :] = part
            else:
                out_ref[:, :] += part

        out_ref[:, :] = jnp.maximum(out_ref[:, :], 0.0)

        ocopy = pltpu.make_async_copy(out_ref, out_hbm, out_sem)
        ocopy.start()
        ocopy.wait()

        for rdma in sends:
            rdma.wait_send()

    return pl.pallas_call(
        body,
        out_shape=jax.ShapeDtypeStruct((m_per, n), jnp.float32),
        in_specs=[
            pl.BlockSpec(memory_space=pltpu.MemorySpace.HBM),
            pl.BlockSpec(memory_space=pltpu.MemorySpace.HBM),
        ],
        out_specs=pl.BlockSpec(memory_space=pltpu.MemorySpace.HBM),
        scratch_shapes=[
            pltpu.VMEM((k_dim, k_per), jnp.float32),
            pltpu.VMEM((N_DEV, half, 2 * k_per), jnp.float32),
            pltpu.VMEM((N_DEV, half, 2 * k_per), jnp.float32),
            pltpu.VMEM((N_DEV, m_per, n), jnp.float32),
            pltpu.VMEM((m_per, n), jnp.float32),
            pltpu.SemaphoreType.DMA((N_DEV,)),
            pltpu.SemaphoreType.DMA((N_DEV,)),
            pltpu.SemaphoreType.DMA((N_DEV,)),
            pltpu.SemaphoreType.DMA,
            pltpu.SemaphoreType.DMA,
        ],
        compiler_params=pltpu.CompilerParams(collective_id=0),
    )(
        pltpu.with_memory_space_constraint(x, pltpu.MemorySpace.HBM),
        pltpu.with_memory_space_constraint(w_mat, pltpu.MemorySpace.HBM),
    )
